# Initial kernel scaffold; baseline (speedup 1.0000x reference)
import jax
import jax.numpy as jnp
from jax.experimental import pallas as pl


def kernel(z, audio_domain, n_q, embedding, proj_W, proj_b):
    raise NotImplementedError("write your pallas kernel here")


# speedup vs baseline: 3.0822x; 3.0822x over previous
import functools

import jax
import jax.numpy as jnp
from jax import lax
from jax.experimental import pallas as pl
from jax.experimental.pallas import tpu as pltpu
from jax.experimental.pallas import tpu_sc as plsc

N_E = 8192
E_DIM = 256
B = 8
H = 1024
CODE_TILE = 512
N_CODE_TILES_WIN = 8
WIN = 2048
_PREC = lax.Precision.DEFAULT


def _project_kernel(emb_ref, w_ref, b_ref, qc_ref):
    e = emb_ref[...]
    w = w_ref[...]
    qc = lax.dot_general(e, w, (((1,), (1,)), ((), ())),
                         preferred_element_type=jnp.float32, precision=_PREC)
    qc_ref[...] = qc + b_ref[...]


def _project(embedding, proj_W, proj_b):
    return pl.pallas_call(
        _project_kernel,
        grid=(N_E // CODE_TILE,),
        in_specs=[
            pl.BlockSpec((CODE_TILE, E_DIM), lambda i: (i, 0)),
            pl.BlockSpec((E_DIM, E_DIM), lambda i: (0, 0)),
            pl.BlockSpec((1, E_DIM), lambda i: (0, 0)),
        ],
        out_specs=pl.BlockSpec((CODE_TILE, E_DIM), lambda i: (i, 0)),
        out_shape=jax.ShapeDtypeStruct((N_E, E_DIM), jnp.float32),
    )(embedding, proj_W, proj_b.reshape(1, E_DIM))


def _argmin_kernel(dom_ref, z_ref, qc_ref, idx_ref, loss_ref, minv_ref, mini_ref):
    b = pl.program_id(0)
    j = pl.program_id(1)
    dom = dom_ref[b]
    active = jnp.logical_or(dom == 2, j < 4)

    @pl.when(jnp.logical_and(b == 0, j == 0))
    def _():
        loss_ref[0, 0] = 0.0

    @pl.when(active)
    def _():
        qc = qc_ref[...]
        zb = z_ref[0]
        prod = lax.dot_general(qc, zb, (((1,), (0,)), ((), ())),
                               preferred_element_type=jnp.float32,
                               precision=_PREC)
        cn = jnp.sum(qc * qc, axis=1, keepdims=True)
        dist = cn - 2.0 * prod
        tmin = jnp.min(dist, axis=0, keepdims=True)
        rows = lax.broadcasted_iota(jnp.int32, (CODE_TILE, H), 0)
        targ = jnp.min(jnp.where(dist == tmin, rows, jnp.int32(2**30)),
                       axis=0, keepdims=True)
        gidx = targ + (dom * WIN + j * CODE_TILE)

        @pl.when(j == 0)
        def _():
            minv_ref[...] = tmin
            mini_ref[...] = gidx

        @pl.when(j > 0)
        def _():
            better = tmin < minv_ref[...]
            minv_ref[...] = jnp.where(better, tmin, minv_ref[...])
            mini_ref[...] = jnp.where(better, gidx, mini_ref[...])

    jlast = jnp.where(dom == 2, 7, 3)

    @pl.when(j == jlast)
    def _():
        zb = z_ref[0]
        loss_ref[0, 0] += jnp.sum(minv_ref[...]) + jnp.sum(zb * zb)

    @pl.when(j == N_CODE_TILES_WIN - 1)
    def _():
        idx_ref[0] = mini_ref[...]

    @pl.when(jnp.logical_and(b == B - 1, j == N_CODE_TILES_WIN - 1))
    def _():
        loss_ref[0, 0] = loss_ref[0, 0] * (1.25 / (B * H * E_DIM))


def _argmin(z, qc, audio_domain):
    grid_spec = pltpu.PrefetchScalarGridSpec(
        num_scalar_prefetch=1,
        grid=(B, N_CODE_TILES_WIN),
        in_specs=[
            pl.BlockSpec((1, E_DIM, H), lambda b, j, dom: (b, 0, 0)),
            pl.BlockSpec((CODE_TILE, E_DIM),
                         lambda b, j, dom: (
                             dom[b] * (WIN // CODE_TILE)
                             + jnp.minimum(j, jnp.where(dom[b] == 2, 7, 3)),
                             0)),
        ],
        out_specs=[
            pl.BlockSpec((1, 1, H), lambda b, j, dom: (b, 0, 0)),
            pl.BlockSpec(block_shape=(1, 1), index_map=lambda b, j, dom: (0, 0),
                         memory_space=pltpu.SMEM),
        ],
        scratch_shapes=[
            pltpu.VMEM((1, H), jnp.float32),
            pltpu.VMEM((1, H), jnp.int32),
        ],
    )
    return pl.pallas_call(
        _argmin_kernel,
        grid_spec=grid_spec,
        out_shape=[
            jax.ShapeDtypeStruct((B, 1, H), jnp.int32),
            jax.ShapeDtypeStruct((1, 1), jnp.float32),
        ],
    )(audio_domain, z, qc)


def _gather_rows(qc, idx):
    n = idx.shape[0]
    nw = 32
    per_w = n // nw
    chunk = 128
    mesh = plsc.VectorSubcoreMesh(core_axis_name="c", subcore_axis_name="s")

    @functools.partial(
        pl.kernel,
        mesh=mesh,
        out_type=jax.ShapeDtypeStruct((n, E_DIM), jnp.float32),
        scratch_types=[
            pltpu.VMEM((per_w,), jnp.int32),
            pltpu.VMEM((per_w, E_DIM), jnp.float32),
            pltpu.SemaphoreType.DMA,
        ],
    )
    def k(table_hbm, idx_hbm, out_hbm, idx_v, rows_v, sem):
        wid = lax.axis_index("s") * 2 + lax.axis_index("c")
        base = wid * per_w
        pltpu.sync_copy(idx_hbm.at[pl.ds(base, per_w)], idx_v)
        copies = []
        for c in range(per_w // chunk):
            copies.append(pltpu.async_copy(
                table_hbm.at[idx_v.at[pl.ds(c * chunk, chunk)]],
                rows_v.at[pl.ds(c * chunk, chunk), :],
                sem))
        for cp in copies:
            cp.wait()
        pltpu.sync_copy(rows_v, out_hbm.at[pl.ds(base, per_w)])

    return k(qc, idx)


def kernel(z, audio_domain, n_q, embedding, proj_W, proj_b):
    del n_q
    dom = audio_domain.astype(jnp.int32)
    qc = _project(embedding, proj_W, proj_b)
    idx3, loss = _argmin(z, qc, dom)
    idx_flat = idx3.reshape(-1)
    zq_rows = _gather_rows(qc, idx_flat)
    z_q = zq_rows.reshape(B, H, E_DIM).transpose(0, 2, 1)
    min_encoding_indices = idx3.reshape(1, B, H)
    return (z_q, min_encoding_indices, loss[0, 0])


# speedup vs baseline: 3.1072x; 1.0081x over previous
import functools

import jax
import jax.numpy as jnp
from jax import lax
from jax.experimental import pallas as pl
from jax.experimental.pallas import tpu as pltpu
from jax.experimental.pallas import tpu_sc as plsc

N_E = 8192
E_DIM = 256
B = 8
H = 1024
CODE_TILE = 512
N_CODE_TILES_WIN = 8
WIN = 2048
_PREC = lax.Precision.DEFAULT


def _project_kernel(emb_ref, w_ref, b_ref, qc_ref):
    e = emb_ref[...]
    w = w_ref[...]
    qc = lax.dot_general(e, w, (((1,), (1,)), ((), ())),
                         preferred_element_type=jnp.float32, precision=_PREC)
    qc_ref[...] = qc + b_ref[...]


def _project(embedding, proj_W, proj_b):
    return pl.pallas_call(
        _project_kernel,
        grid=(N_E // CODE_TILE,),
        in_specs=[
            pl.BlockSpec((CODE_TILE, E_DIM), lambda i: (i, 0)),
            pl.BlockSpec((E_DIM, E_DIM), lambda i: (0, 0)),
            pl.BlockSpec((1, E_DIM), lambda i: (0, 0)),
        ],
        out_specs=pl.BlockSpec((CODE_TILE, E_DIM), lambda i: (i, 0)),
        out_shape=jax.ShapeDtypeStruct((N_E, E_DIM), jnp.float32),
    )(embedding, proj_W, proj_b.reshape(1, E_DIM))


def _argmin_kernel(dom_ref, z_ref, qc_ref, idx_ref, loss_ref,
                   minv_ref, mini_ref, zn_ref):
    b = pl.program_id(0)
    j = pl.program_id(1)
    dom = dom_ref[b]
    active = jnp.logical_or(dom == 2, j < 4)

    @pl.when(jnp.logical_and(b == 0, j == 0))
    def _():
        loss_ref[0, 0] = 0.0

    @pl.when(j == 0)
    def _():
        zb = z_ref[0]
        zn_ref[...] = jnp.sum(zb * zb, axis=0, keepdims=True)

    @pl.when(active)
    def _():
        qc = qc_ref[...]
        zb = z_ref[0]
        prodm2 = lax.dot_general(qc * (-2.0), zb, (((1,), (0,)), ((), ())),
                                 preferred_element_type=jnp.float32,
                                 precision=_PREC)
        cn = jnp.sum(qc * qc, axis=1, keepdims=True)
        dist = (zn_ref[...] + cn) + prodm2
        tmin = jnp.min(dist, axis=0, keepdims=True)
        rows = lax.broadcasted_iota(jnp.int32, (CODE_TILE, H), 0)
        targ = jnp.min(jnp.where(dist == tmin, rows, jnp.int32(2**30)),
                       axis=0, keepdims=True)
        gidx = targ + (dom * WIN + j * CODE_TILE)

        @pl.when(j == 0)
        def _():
            minv_ref[...] = tmin
            mini_ref[...] = gidx

        @pl.when(j > 0)
        def _():
            better = tmin < minv_ref[...]
            minv_ref[...] = jnp.where(better, tmin, minv_ref[...])
            mini_ref[...] = jnp.where(better, gidx, mini_ref[...])

    jlast = jnp.where(dom == 2, 7, 3)

    @pl.when(j == jlast)
    def _():
        loss_ref[0, 0] += jnp.sum(minv_ref[...])

    @pl.when(j == N_CODE_TILES_WIN - 1)
    def _():
        idx_ref[0] = mini_ref[...]

    @pl.when(jnp.logical_and(b == B - 1, j == N_CODE_TILES_WIN - 1))
    def _():
        loss_ref[0, 0] = loss_ref[0, 0] * (1.25 / (B * H * E_DIM))


def _argmin(z, qc, audio_domain):
    grid_spec = pltpu.PrefetchScalarGridSpec(
        num_scalar_prefetch=1,
        grid=(B, N_CODE_TILES_WIN),
        in_specs=[
            pl.BlockSpec((1, E_DIM, H), lambda b, j, dom: (b, 0, 0)),
            pl.BlockSpec((CODE_TILE, E_DIM),
                         lambda b, j, dom: (
                             dom[b] * (WIN // CODE_TILE)
                             + jnp.minimum(j, jnp.where(dom[b] == 2, 7, 3)),
                             0)),
        ],
        out_specs=[
            pl.BlockSpec((1, 1, H), lambda b, j, dom: (b, 0, 0)),
            pl.BlockSpec(block_shape=(1, 1), index_map=lambda b, j, dom: (0, 0),
                         memory_space=pltpu.SMEM),
        ],
        scratch_shapes=[
            pltpu.VMEM((1, H), jnp.float32),
            pltpu.VMEM((1, H), jnp.int32),
            pltpu.VMEM((1, H), jnp.float32),
        ],
    )
    return pl.pallas_call(
        _argmin_kernel,
        grid_spec=grid_spec,
        out_shape=[
            jax.ShapeDtypeStruct((B, 1, H), jnp.int32),
            jax.ShapeDtypeStruct((1, 1), jnp.float32),
        ],
    )(audio_domain, z, qc)


def _gather_rows(qc, idx):
    n = idx.shape[0]
    nw = 32
    per_w = n // nw
    chunk = 128
    mesh = plsc.VectorSubcoreMesh(core_axis_name="c", subcore_axis_name="s")

    @functools.partial(
        pl.kernel,
        mesh=mesh,
        out_type=jax.ShapeDtypeStruct((n, E_DIM), jnp.float32),
        scratch_types=[
            pltpu.VMEM((per_w,), jnp.int32),
            pltpu.VMEM((per_w, E_DIM), jnp.float32),
            pltpu.SemaphoreType.DMA,
        ],
    )
    def k(table_hbm, idx_hbm, out_hbm, idx_v, rows_v, sem):
        wid = lax.axis_index("s") * 2 + lax.axis_index("c")
        base = wid * per_w
        pltpu.sync_copy(idx_hbm.at[pl.ds(base, per_w)], idx_v)
        copies = []
        for c in range(per_w // chunk):
            copies.append(pltpu.async_copy(
                table_hbm.at[idx_v.at[pl.ds(c * chunk, chunk)]],
                rows_v.at[pl.ds(c * chunk, chunk), :],
                sem))
        for cp in copies:
            cp.wait()
        pltpu.sync_copy(rows_v, out_hbm.at[pl.ds(base, per_w)])

    return k(qc, idx)


def kernel(z, audio_domain, n_q, embedding, proj_W, proj_b):
    del n_q
    dom = audio_domain.astype(jnp.int32)
    qc = _project(embedding, proj_W, proj_b)
    idx3, loss = _argmin(z, qc, dom)
    idx_flat = idx3.reshape(-1)
    zq_rows = _gather_rows(qc, idx_flat)
    z_q = zq_rows.reshape(B, H, E_DIM).transpose(0, 2, 1)
    min_encoding_indices = idx3.reshape(1, B, H)
    return (z_q, min_encoding_indices, loss[0, 0])


# speedup vs baseline: 3.4827x; 1.1209x over previous
import functools

import jax
import jax.numpy as jnp
from jax import lax
from jax.experimental import pallas as pl
from jax.experimental.pallas import tpu as pltpu
from jax.experimental.pallas import tpu_sc as plsc

N_E = 8192
E_DIM = 256
B = 8
H = 1024
CODE_TILE = 1024
N_CODE_TILES_WIN = 4096 // CODE_TILE
HALF_TILES = 2048 // CODE_TILE
WIN = 2048
_PREC = lax.Precision.DEFAULT


def _project_kernel(emb_ref, w_ref, b_ref, qc_ref):
    e = emb_ref[...]
    w = w_ref[...]
    qc = lax.dot_general(e, w, (((1,), (1,)), ((), ())),
                         preferred_element_type=jnp.float32, precision=_PREC)
    qc_ref[...] = qc + b_ref[...]


def _project(embedding, proj_W, proj_b):
    return pl.pallas_call(
        _project_kernel,
        grid=(N_E // CODE_TILE,),
        in_specs=[
            pl.BlockSpec((CODE_TILE, E_DIM), lambda i: (i, 0)),
            pl.BlockSpec((E_DIM, E_DIM), lambda i: (0, 0)),
            pl.BlockSpec((1, E_DIM), lambda i: (0, 0)),
        ],
        out_specs=pl.BlockSpec((CODE_TILE, E_DIM), lambda i: (i, 0)),
        out_shape=jax.ShapeDtypeStruct((N_E, E_DIM), jnp.float32),
    )(embedding, proj_W, proj_b.reshape(1, E_DIM))


def _argmin_kernel(dom_ref, z_ref, qc_ref, idx_ref, loss_ref,
                   minv_ref, mini_ref, zn_ref):
    b = pl.program_id(0)
    j = pl.program_id(1)
    dom = dom_ref[b]
    active = jnp.logical_or(dom == 2, j < HALF_TILES)

    @pl.when(jnp.logical_and(b == 0, j == 0))
    def _():
        loss_ref[0, 0] = 0.0

    @pl.when(j == 0)
    def _():
        zb = z_ref[0]
        zn_ref[...] = jnp.sum(zb * zb, axis=0, keepdims=True)

    @pl.when(active)
    def _():
        qc = qc_ref[...]
        zb = z_ref[0]
        prodm2 = lax.dot_general(qc * (-2.0), zb, (((1,), (0,)), ((), ())),
                                 preferred_element_type=jnp.float32,
                                 precision=_PREC)
        cn = jnp.sum(qc * qc, axis=1, keepdims=True)
        dist = (zn_ref[...] + cn) + prodm2
        tmin = jnp.min(dist, axis=0, keepdims=True)
        rows = lax.broadcasted_iota(jnp.int32, (CODE_TILE, H), 0)
        targ = jnp.min(jnp.where(dist == tmin, rows, jnp.int32(2**30)),
                       axis=0, keepdims=True)
        gidx = targ + (dom * WIN + j * CODE_TILE)

        @pl.when(j == 0)
        def _():
            minv_ref[...] = tmin
            mini_ref[...] = gidx

        @pl.when(j > 0)
        def _():
            better = tmin < minv_ref[...]
            minv_ref[...] = jnp.where(better, tmin, minv_ref[...])
            mini_ref[...] = jnp.where(better, gidx, mini_ref[...])

    jlast = jnp.where(dom == 2, N_CODE_TILES_WIN - 1, HALF_TILES - 1)

    @pl.when(j == jlast)
    def _():
        loss_ref[0, 0] += jnp.sum(minv_ref[...])

    @pl.when(j == N_CODE_TILES_WIN - 1)
    def _():
        idx_ref[0] = mini_ref[...]

    @pl.when(jnp.logical_and(b == B - 1, j == N_CODE_TILES_WIN - 1))
    def _():
        loss_ref[0, 0] = loss_ref[0, 0] * (1.25 / (B * H * E_DIM))


def _argmin(z, qc, audio_domain):
    grid_spec = pltpu.PrefetchScalarGridSpec(
        num_scalar_prefetch=1,
        grid=(B, N_CODE_TILES_WIN),
        in_specs=[
            pl.BlockSpec((1, E_DIM, H), lambda b, j, dom: (b, 0, 0)),
            pl.BlockSpec((CODE_TILE, E_DIM),
                         lambda b, j, dom: (
                             dom[b] * (WIN // CODE_TILE)
                             + jnp.minimum(j, jnp.where(dom[b] == 2,
                                                       N_CODE_TILES_WIN - 1,
                                                       HALF_TILES - 1)),
                             0)),
        ],
        out_specs=[
            pl.BlockSpec((1, 1, H), lambda b, j, dom: (b, 0, 0)),
            pl.BlockSpec(block_shape=(1, 1), index_map=lambda b, j, dom: (0, 0),
                         memory_space=pltpu.SMEM),
        ],
        scratch_shapes=[
            pltpu.VMEM((1, H), jnp.float32),
            pltpu.VMEM((1, H), jnp.int32),
            pltpu.VMEM((1, H), jnp.float32),
        ],
    )
    return pl.pallas_call(
        _argmin_kernel,
        grid_spec=grid_spec,
        out_shape=[
            jax.ShapeDtypeStruct((B, 1, H), jnp.int32),
            jax.ShapeDtypeStruct((1, 1), jnp.float32),
        ],
    )(audio_domain, z, qc)


def _gather_rows(qc, idx):
    n = idx.shape[0]
    nw = 32
    per_w = n // nw
    chunk = 128
    mesh = plsc.VectorSubcoreMesh(core_axis_name="c", subcore_axis_name="s")

    @functools.partial(
        pl.kernel,
        mesh=mesh,
        out_type=jax.ShapeDtypeStruct((n, E_DIM), jnp.float32),
        scratch_types=[
            pltpu.VMEM((per_w,), jnp.int32),
            pltpu.VMEM((per_w, E_DIM), jnp.float32),
            pltpu.SemaphoreType.DMA,
        ],
    )
    def k(table_hbm, idx_hbm, out_hbm, idx_v, rows_v, sem):
        wid = lax.axis_index("s") * 2 + lax.axis_index("c")
        base = wid * per_w
        pltpu.sync_copy(idx_hbm.at[pl.ds(base, per_w)], idx_v)
        copies = []
        for c in range(per_w // chunk):
            copies.append(pltpu.async_copy(
                table_hbm.at[idx_v.at[pl.ds(c * chunk, chunk)]],
                rows_v.at[pl.ds(c * chunk, chunk), :],
                sem))
        for cp in copies:
            cp.wait()
        pltpu.sync_copy(rows_v, out_hbm.at[pl.ds(base, per_w)])

    return k(qc, idx)


def kernel(z, audio_domain, n_q, embedding, proj_W, proj_b):
    del n_q
    dom = audio_domain.astype(jnp.int32)
    qc = _project(embedding, proj_W, proj_b)
    idx3, loss = _argmin(z, qc, dom)
    idx_flat = idx3.reshape(-1)
    zq_rows = _gather_rows(qc, idx_flat)
    z_q = zq_rows.reshape(B, H, E_DIM).transpose(0, 2, 1)
    min_encoding_indices = idx3.reshape(1, B, H)
    return (z_q, min_encoding_indices, loss[0, 0])


# speedup vs baseline: 4.0343x; 1.1584x over previous
import functools

import jax
import jax.numpy as jnp
from jax import lax
from jax.experimental import pallas as pl
from jax.experimental.pallas import tpu as pltpu
from jax.experimental.pallas import tpu_sc as plsc

N_E = 8192
E_DIM = 256
B = 8
H = 1024
CODE_TILE = 2048
N_CODE_TILES_WIN = 4096 // CODE_TILE
HALF_TILES = 2048 // CODE_TILE
WIN = 2048
_PREC = lax.Precision.DEFAULT


def _project_kernel(emb_ref, w_ref, b_ref, qc_ref):
    e = emb_ref[...]
    w = w_ref[...]
    qc = lax.dot_general(e, w, (((1,), (1,)), ((), ())),
                         preferred_element_type=jnp.float32, precision=_PREC)
    qc_ref[...] = qc + b_ref[...]


def _project(embedding, proj_W, proj_b):
    return pl.pallas_call(
        _project_kernel,
        grid=(N_E // CODE_TILE,),
        in_specs=[
            pl.BlockSpec((CODE_TILE, E_DIM), lambda i: (i, 0)),
            pl.BlockSpec((E_DIM, E_DIM), lambda i: (0, 0)),
            pl.BlockSpec((1, E_DIM), lambda i: (0, 0)),
        ],
        out_specs=pl.BlockSpec((CODE_TILE, E_DIM), lambda i: (i, 0)),
        out_shape=jax.ShapeDtypeStruct((N_E, E_DIM), jnp.float32),
    )(embedding, proj_W, proj_b.reshape(1, E_DIM))


def _argmin_kernel(dom_ref, z_ref, qc_ref, idx_ref, loss_ref,
                   minv_ref, mini_ref, zn_ref):
    b = pl.program_id(0)
    j = pl.program_id(1)
    dom = dom_ref[b]
    active = jnp.logical_or(dom == 2, j < HALF_TILES)

    @pl.when(jnp.logical_and(b == 0, j == 0))
    def _():
        loss_ref[0, 0] = 0.0

    @pl.when(j == 0)
    def _():
        zb = z_ref[0]
        zn_ref[...] = jnp.sum(zb * zb, axis=0, keepdims=True)

    @pl.when(active)
    def _():
        qc = qc_ref[...]
        zb = z_ref[0]
        prodm2 = lax.dot_general(qc * (-2.0), zb, (((1,), (0,)), ((), ())),
                                 preferred_element_type=jnp.float32,
                                 precision=_PREC)
        cn = jnp.sum(qc * qc, axis=1, keepdims=True)
        dist = (zn_ref[...] + cn) + prodm2
        tmin = jnp.min(dist, axis=0, keepdims=True)
        targ = jnp.argmin(dist, axis=0).reshape(1, H)
        gidx = targ + (dom * WIN + j * CODE_TILE)

        @pl.when(j == 0)
        def _():
            minv_ref[...] = tmin
            mini_ref[...] = gidx

        @pl.when(j > 0)
        def _():
            better = tmin < minv_ref[...]
            minv_ref[...] = jnp.where(better, tmin, minv_ref[...])
            mini_ref[...] = jnp.where(better, gidx, mini_ref[...])

    jlast = jnp.where(dom == 2, N_CODE_TILES_WIN - 1, HALF_TILES - 1)

    @pl.when(j == jlast)
    def _():
        loss_ref[0, 0] += jnp.sum(minv_ref[...])

    @pl.when(j == N_CODE_TILES_WIN - 1)
    def _():
        idx_ref[0] = mini_ref[...]

    @pl.when(jnp.logical_and(b == B - 1, j == N_CODE_TILES_WIN - 1))
    def _():
        loss_ref[0, 0] = loss_ref[0, 0] * (1.25 / (B * H * E_DIM))


def _code_tile_map(b, j, dom):
    return (dom[b] * (WIN // CODE_TILE)
            + jnp.minimum(j, jnp.where(dom[b] == 2,
                                       N_CODE_TILES_WIN - 1,
                                       HALF_TILES - 1)),
            0)


def _argmin(z, qc, audio_domain):
    grid_spec = pltpu.PrefetchScalarGridSpec(
        num_scalar_prefetch=1,
        grid=(B, N_CODE_TILES_WIN),
        in_specs=[
            pl.BlockSpec((1, E_DIM, H), lambda b, j, dom: (b, 0, 0)),
            pl.BlockSpec((CODE_TILE, E_DIM), _code_tile_map),
        ],
        out_specs=[
            pl.BlockSpec((1, 1, H), lambda b, j, dom: (b, 0, 0)),
            pl.BlockSpec(block_shape=(1, 1), index_map=lambda b, j, dom: (0, 0),
                         memory_space=pltpu.SMEM),
        ],
        scratch_shapes=[
            pltpu.VMEM((1, H), jnp.float32),
            pltpu.VMEM((1, H), jnp.int32),
            pltpu.VMEM((1, H), jnp.float32),
        ],
    )
    return pl.pallas_call(
        _argmin_kernel,
        grid_spec=grid_spec,
        out_shape=[
            jax.ShapeDtypeStruct((B, 1, H), jnp.int32),
            jax.ShapeDtypeStruct((1, 1), jnp.float32),
        ],
    )(audio_domain, z, qc)


def _gather_rows(qc, idx):
    n = idx.shape[0]
    nw = 32
    per_w = n // nw
    chunk = 128
    mesh = plsc.VectorSubcoreMesh(core_axis_name="c", subcore_axis_name="s")

    @functools.partial(
        pl.kernel,
        mesh=mesh,
        out_type=jax.ShapeDtypeStruct((n, E_DIM), jnp.float32),
        scratch_types=[
            pltpu.VMEM((per_w,), jnp.int32),
            pltpu.VMEM((per_w, E_DIM), jnp.float32),
            pltpu.SemaphoreType.DMA,
        ],
    )
    def k(table_hbm, idx_hbm, out_hbm, idx_v, rows_v, sem):
        wid = lax.axis_index("s") * 2 + lax.axis_index("c")
        base = wid * per_w
        pltpu.sync_copy(idx_hbm.at[pl.ds(base, per_w)], idx_v)
        copies = []
        for c in range(per_w // chunk):
            copies.append(pltpu.async_copy(
                table_hbm.at[idx_v.at[pl.ds(c * chunk, chunk)]],
                rows_v.at[pl.ds(c * chunk, chunk), :],
                sem))
        for cp in copies:
            cp.wait()
        pltpu.sync_copy(rows_v, out_hbm.at[pl.ds(base, per_w)])

    return k(qc, idx)


def kernel(z, audio_domain, n_q, embedding, proj_W, proj_b):
    del n_q
    dom = audio_domain.astype(jnp.int32)
    qc = _project(embedding, proj_W, proj_b)
    idx3, loss = _argmin(z, qc, dom)
    idx_flat = idx3.reshape(-1)
    zq_rows = _gather_rows(qc, idx_flat)
    z_q = zq_rows.reshape(B, H, E_DIM).transpose(0, 2, 1)
    min_encoding_indices = idx3.reshape(1, B, H)
    return (z_q, min_encoding_indices, loss[0, 0])


# speedup vs baseline: 4.2410x; 1.0512x over previous
import functools

import jax
import jax.numpy as jnp
from jax import lax
from jax.experimental import pallas as pl
from jax.experimental.pallas import tpu as pltpu
from jax.experimental.pallas import tpu_sc as plsc

N_E = 8192
E_DIM = 256
B = 8
H = 1024
CODE_TILE = 2048
NPROJ = N_E // CODE_TILE
NWIN = 4096 // CODE_TILE
WIN = 2048
_PREC = lax.Precision.DEFAULT


def _fused_kernel(dom_ref, emb_ref, w_ref, b_ref, z_ref,
                  qc_out_ref, idx_ref, loss_ref,
                  qc_scr, minv_ref, mini_ref, zn_ref):
    t = pl.program_id(0)
    s = jnp.maximum(t - NPROJ, 0)
    b = s // NWIN
    j = s % NWIN
    dom = dom_ref[b]
    phase2 = t >= NPROJ

    @pl.when(t < NPROJ)
    def _():
        e = emb_ref[...]
        qc = lax.dot_general(e, w_ref[...], (((1,), (1,)), ((), ())),
                             preferred_element_type=jnp.float32,
                             precision=_PREC)
        qc = qc + b_ref[...]
        qc_out_ref[...] = qc
        qc_scr[pl.ds(t * CODE_TILE, CODE_TILE), :] = qc

    @pl.when(t == NPROJ)
    def _():
        loss_ref[0, 0] = 0.0

    @pl.when(jnp.logical_and(phase2, j == 0))
    def _():
        zb = z_ref[0]
        zn_ref[...] = jnp.sum(zb * zb, axis=0, keepdims=True)

    active = jnp.logical_and(phase2, jnp.logical_or(dom == 2, j == 0))

    @pl.when(active)
    def _():
        w_tile = dom + jnp.minimum(j, jnp.where(dom == 2, NWIN - 1, 0))
        qc = qc_scr[pl.ds(w_tile * CODE_TILE, CODE_TILE), :]
        zb = z_ref[0]
        prodm2 = lax.dot_general(qc * (-2.0), zb, (((1,), (0,)), ((), ())),
                                 preferred_element_type=jnp.float32,
                                 precision=_PREC)
        cn = jnp.sum(qc * qc, axis=1, keepdims=True)
        dist = (zn_ref[...] + cn) + prodm2
        tmin = jnp.min(dist, axis=0, keepdims=True)
        targ = jnp.argmin(dist, axis=0).reshape(1, H)
        gidx = targ + (dom * WIN + j * CODE_TILE)

        @pl.when(j == 0)
        def _():
            minv_ref[...] = tmin
            mini_ref[...] = gidx

        @pl.when(j > 0)
        def _():
            better = tmin < minv_ref[...]
            minv_ref[...] = jnp.where(better, tmin, minv_ref[...])
            mini_ref[...] = jnp.where(better, gidx, mini_ref[...])

    jlast = jnp.where(dom == 2, NWIN - 1, 0)

    @pl.when(jnp.logical_and(phase2, j == jlast))
    def _():
        loss_ref[0, 0] += jnp.sum(minv_ref[...])

    @pl.when(jnp.logical_and(phase2, j == NWIN - 1))
    def _():
        idx_ref[0] = mini_ref[...]

    @pl.when(t == NPROJ + B * NWIN - 1)
    def _():
        loss_ref[0, 0] = loss_ref[0, 0] * (1.25 / (B * H * E_DIM))


def _project_and_argmin(z, embedding, proj_W, proj_b, audio_domain):
    grid_spec = pltpu.PrefetchScalarGridSpec(
        num_scalar_prefetch=1,
        grid=(NPROJ + B * NWIN,),
        in_specs=[
            pl.BlockSpec((CODE_TILE, E_DIM),
                         lambda t, dom: (jnp.minimum(t, NPROJ - 1), 0)),
            pl.BlockSpec((E_DIM, E_DIM), lambda t, dom: (0, 0)),
            pl.BlockSpec((1, E_DIM), lambda t, dom: (0, 0)),
            pl.BlockSpec((1, E_DIM, H),
                         lambda t, dom: (jnp.maximum(t - NPROJ, 0) // NWIN, 0, 0)),
        ],
        out_specs=[
            pl.BlockSpec((CODE_TILE, E_DIM),
                         lambda t, dom: (jnp.minimum(t, NPROJ - 1), 0)),
            pl.BlockSpec((1, 1, H),
                         lambda t, dom: (jnp.maximum(t - NPROJ, 0) // NWIN, 0, 0)),
            pl.BlockSpec(block_shape=(1, 1), index_map=lambda t, dom: (0, 0),
                         memory_space=pltpu.SMEM),
        ],
        scratch_shapes=[
            pltpu.VMEM((N_E, E_DIM), jnp.float32),
            pltpu.VMEM((1, H), jnp.float32),
            pltpu.VMEM((1, H), jnp.int32),
            pltpu.VMEM((1, H), jnp.float32),
        ],
    )
    return pl.pallas_call(
        _fused_kernel,
        grid_spec=grid_spec,
        out_shape=[
            jax.ShapeDtypeStruct((N_E, E_DIM), jnp.float32),
            jax.ShapeDtypeStruct((B, 1, H), jnp.int32),
            jax.ShapeDtypeStruct((1, 1), jnp.float32),
        ],
    )(audio_domain, embedding, proj_W, proj_b.reshape(1, E_DIM), z)


def _gather_rows(qc, idx):
    n = idx.shape[0]
    nw = 32
    per_w = n // nw
    chunk = 128
    mesh = plsc.VectorSubcoreMesh(core_axis_name="c", subcore_axis_name="s")

    @functools.partial(
        pl.kernel,
        mesh=mesh,
        out_type=jax.ShapeDtypeStruct((n, E_DIM), jnp.float32),
        scratch_types=[
            pltpu.VMEM((per_w,), jnp.int32),
            pltpu.VMEM((per_w, E_DIM), jnp.float32),
            pltpu.SemaphoreType.DMA,
        ],
    )
    def k(table_hbm, idx_hbm, out_hbm, idx_v, rows_v, sem):
        wid = lax.axis_index("s") * 2 + lax.axis_index("c")
        base = wid * per_w
        pltpu.sync_copy(idx_hbm.at[pl.ds(base, per_w)], idx_v)
        copies = []
        for c in range(per_w // chunk):
            copies.append(pltpu.async_copy(
                table_hbm.at[idx_v.at[pl.ds(c * chunk, chunk)]],
                rows_v.at[pl.ds(c * chunk, chunk), :],
                sem))
        for cp in copies:
            cp.wait()
        pltpu.sync_copy(rows_v, out_hbm.at[pl.ds(base, per_w)])

    return k(qc, idx)


def kernel(z, audio_domain, n_q, embedding, proj_W, proj_b):
    del n_q
    dom = audio_domain.astype(jnp.int32)
    qc, idx3, loss = _project_and_argmin(z, embedding, proj_W, proj_b, dom)
    idx_flat = idx3.reshape(-1)
    zq_rows = _gather_rows(qc, idx_flat)
    z_q = zq_rows.reshape(B, H, E_DIM).transpose(0, 2, 1)
    min_encoding_indices = idx3.reshape(1, B, H)
    return (z_q, min_encoding_indices, loss[0, 0])


# speedup vs baseline: 4.2811x; 1.0095x over previous
import functools

import jax
import jax.numpy as jnp
from jax import lax
from jax.experimental import pallas as pl
from jax.experimental.pallas import tpu as pltpu
from jax.experimental.pallas import tpu_sc as plsc

N_E = 8192
E_DIM = 256
B = 8
H = 1024
CODE_TILE = 2048
NPROJ = N_E // CODE_TILE
NWIN = 4096 // CODE_TILE
WIN = 2048
_PREC = lax.Precision.DEFAULT


def _fused_kernel(dom_ref, emb_ref, w_ref, b_ref, z_ref,
                  qc_out_ref, idx_ref, loss_ref,
                  qc_scr, minv_ref, mini_ref, zn_ref):
    t = pl.program_id(0)
    s = jnp.maximum(t - NPROJ, 0)
    b = s // NWIN
    j = s % NWIN
    dom = dom_ref[b]
    phase2 = t >= NPROJ

    @pl.when(t < NPROJ)
    def _():
        e = emb_ref[...]
        qc = lax.dot_general(e, w_ref[...], (((1,), (1,)), ((), ())),
                             preferred_element_type=jnp.float32,
                             precision=_PREC)
        qc = qc + b_ref[...]
        qc_out_ref[...] = qc
        qc_scr[pl.ds(t * CODE_TILE, CODE_TILE), :] = qc * (-2.0)

    @pl.when(t == NPROJ)
    def _():
        loss_ref[0, 0] = 0.0

    @pl.when(jnp.logical_and(phase2, j == 0))
    def _():
        zb = z_ref[0]
        zn_ref[...] = jnp.sum(zb * zb, axis=0, keepdims=True)

    active = jnp.logical_and(phase2, jnp.logical_or(dom == 2, j == 0))

    @pl.when(active)
    def _():
        w_tile = dom + jnp.minimum(j, jnp.where(dom == 2, NWIN - 1, 0))
        qcm2 = qc_scr[pl.ds(w_tile * CODE_TILE, CODE_TILE), :]
        zb = z_ref[0]
        prodm2 = lax.dot_general(qcm2, zb, (((1,), (0,)), ((), ())),
                                 preferred_element_type=jnp.float32,
                                 precision=_PREC)
        cn = jnp.sum(qcm2 * qcm2, axis=1, keepdims=True) * 0.25
        dist = (zn_ref[...] + cn) + prodm2
        tmin = jnp.min(dist, axis=0, keepdims=True)
        targ = jnp.argmin(dist, axis=0).reshape(1, H)
        gidx = targ + (dom * WIN + j * CODE_TILE)

        @pl.when(j == 0)
        def _():
            minv_ref[...] = tmin
            mini_ref[...] = gidx

        @pl.when(j > 0)
        def _():
            better = tmin < minv_ref[...]
            minv_ref[...] = jnp.where(better, tmin, minv_ref[...])
            mini_ref[...] = jnp.where(better, gidx, mini_ref[...])

    jlast = jnp.where(dom == 2, NWIN - 1, 0)

    @pl.when(jnp.logical_and(phase2, j == jlast))
    def _():
        loss_ref[0, 0] += jnp.sum(minv_ref[...])

    @pl.when(jnp.logical_and(phase2, j == NWIN - 1))
    def _():
        idx_ref[0] = mini_ref[...]

    @pl.when(t == NPROJ + B * NWIN - 1)
    def _():
        loss_ref[0, 0] = loss_ref[0, 0] * (1.25 / (B * H * E_DIM))


def _project_and_argmin(z, embedding, proj_W, proj_b, audio_domain):
    grid_spec = pltpu.PrefetchScalarGridSpec(
        num_scalar_prefetch=1,
        grid=(NPROJ + B * NWIN,),
        in_specs=[
            pl.BlockSpec((CODE_TILE, E_DIM),
                         lambda t, dom: (jnp.minimum(t, NPROJ - 1), 0)),
            pl.BlockSpec((E_DIM, E_DIM), lambda t, dom: (0, 0)),
            pl.BlockSpec((1, E_DIM), lambda t, dom: (0, 0)),
            pl.BlockSpec((1, E_DIM, H),
                         lambda t, dom: (jnp.maximum(t - NPROJ, 0) // NWIN, 0, 0)),
        ],
        out_specs=[
            pl.BlockSpec((CODE_TILE, E_DIM),
                         lambda t, dom: (jnp.minimum(t, NPROJ - 1), 0)),
            pl.BlockSpec((1, 1, H),
                         lambda t, dom: (jnp.maximum(t - NPROJ, 0) // NWIN, 0, 0)),
            pl.BlockSpec(block_shape=(1, 1), index_map=lambda t, dom: (0, 0),
                         memory_space=pltpu.SMEM),
        ],
        scratch_shapes=[
            pltpu.VMEM((N_E, E_DIM), jnp.float32),
            pltpu.VMEM((1, H), jnp.float32),
            pltpu.VMEM((1, H), jnp.int32),
            pltpu.VMEM((1, H), jnp.float32),
        ],
    )
    return pl.pallas_call(
        _fused_kernel,
        grid_spec=grid_spec,
        out_shape=[
            jax.ShapeDtypeStruct((N_E, E_DIM), jnp.float32),
            jax.ShapeDtypeStruct((B, 1, H), jnp.int32),
            jax.ShapeDtypeStruct((1, 1), jnp.float32),
        ],
    )(audio_domain, embedding, proj_W, proj_b.reshape(1, E_DIM), z)


def _gather_rows(qc, idx):
    n = idx.shape[0]
    nw = 32
    per_w = n // nw
    chunk = 128
    mesh = plsc.VectorSubcoreMesh(core_axis_name="c", subcore_axis_name="s")

    @functools.partial(
        pl.kernel,
        mesh=mesh,
        out_type=jax.ShapeDtypeStruct((n, E_DIM), jnp.float32),
        scratch_types=[
            pltpu.VMEM((per_w,), jnp.int32),
            pltpu.VMEM((per_w, E_DIM), jnp.float32),
            pltpu.SemaphoreType.DMA,
        ],
    )
    def k(table_hbm, idx_hbm, out_hbm, idx_v, rows_v, sem):
        wid = lax.axis_index("s") * 2 + lax.axis_index("c")
        base = wid * per_w
        pltpu.sync_copy(idx_hbm.at[pl.ds(base, per_w)], idx_v)
        copies = []
        for c in range(per_w // chunk):
            copies.append(pltpu.async_copy(
                table_hbm.at[idx_v.at[pl.ds(c * chunk, chunk)]],
                rows_v.at[pl.ds(c * chunk, chunk), :],
                sem))
        for cp in copies:
            cp.wait()
        pltpu.sync_copy(rows_v, out_hbm.at[pl.ds(base, per_w)])

    return k(qc, idx)


def kernel(z, audio_domain, n_q, embedding, proj_W, proj_b):
    del n_q
    dom = audio_domain.astype(jnp.int32)
    qc, idx3, loss = _project_and_argmin(z, embedding, proj_W, proj_b, dom)
    idx_flat = idx3.reshape(-1)
    zq_rows = _gather_rows(qc, idx_flat)
    z_q = zq_rows.reshape(B, H, E_DIM).transpose(0, 2, 1)
    min_encoding_indices = idx3.reshape(1, B, H)
    return (z_q, min_encoding_indices, loss[0, 0])
